# baseline (speedup 1.0000x reference)
import jax
import jax.numpy as jnp
from jax.experimental import pallas as pl
from jax.experimental.pallas import tpu as pltpu

_NUM_BET_BINS = 32
_D = 128
_NUM_STREETS = 4
_OFFSET = 10
_PACKED_ROWS = 40
_SEG = 56


def _fused_kernel(tok_ref, act_ref, st_ref, x_ref, w_ref, b_ref, g_ref,
                  be_ref, t_ref, out_ref):
    rb, ll = tok_ref.shape
    p = rb * _SEG

    zpad = jnp.zeros((_SEG - ll, _NUM_BET_BINS), jnp.float32)
    xs = []
    for i in range(rb):
        xs.append(x_ref[i])
        xs.append(zpad)
    x = jnp.concatenate(xs, axis=0)
    h = jnp.dot(x, w_ref[...], preferred_element_type=jnp.float32)
    h = h + b_ref[...]
    mu = jnp.mean(h, axis=-1, keepdims=True)
    d = h - mu
    var = jnp.mean(d * d, axis=-1, keepdims=True)
    hn = d * jax.lax.rsqrt(var + 1e-5) * g_ref[...] + be_ref[...]
    hr = jnp.maximum(hn, 0.0)

    ipad = jnp.zeros((1, _SEG - ll), jnp.int32)
    cat = lambda r: jnp.concatenate(
        [q for i in range(rb) for q in (r[i:i + 1, :], ipad)], axis=1)
    tok = cat(tok_ref)
    mask = (tok >= _OFFSET) & (tok < _OFFSET + _NUM_BET_BINS)
    a = jnp.where(mask, jnp.clip(cat(act_ref), 0, 1), -1)
    s = jnp.where(mask, jnp.clip(cat(st_ref), 0, _NUM_STREETS - 1) + 2, -1)
    t = jnp.where(mask, jnp.clip(tok - _OFFSET, 0, _NUM_BET_BINS - 1) + 6, -1)

    sub = jax.lax.broadcasted_iota(jnp.int32, (_PACKED_ROWS, p), 0)
    oh = (jnp.where(sub == a, 1.0, 0.0)
          + jnp.where(sub == s, 1.0, 0.0)
          + jnp.where(sub == t, 1.0, 0.0))
    ea = jax.lax.dot_general(oh, t_ref[...], (((0,), (0,)), ((), ())),
                             preferred_element_type=jnp.float32)
    out = ea[:, :_D] + ea[:, _D:] * hr
    for i in range(rb):
        out_ref[i] = out[i * _SEG:i * _SEG + ll, :]


def kernel(token_ids, action_actors, action_streets, action_legal_masks,
           actor_emb_w, street_emb_w, action_type_emb_w, mlp_w, mlp_b,
           ln_gamma, ln_beta):
    B, L = token_ids.shape
    RB = 256
    num_blocks = pl.cdiv(B, RB)

    tok = token_ids.astype(jnp.int32)
    act = action_actors.astype(jnp.int32)
    st = action_streets.astype(jnp.int32)

    packed = jnp.concatenate([
        actor_emb_w, street_emb_w, action_type_emb_w,
        jnp.zeros((_PACKED_ROWS - 38, _D), jnp.float32)], axis=0)
    ind = jnp.zeros((_PACKED_ROWS, 1), jnp.float32).at[0:2, 0].set(1.0)
    packed = jnp.concatenate([packed, ind], axis=1)

    idx_spec = pl.BlockSpec((RB, L), lambda i: (i, 0))
    full_spec = lambda shape: pl.BlockSpec(shape, lambda i: (0,) * len(shape))

    out = pl.pallas_call(
        _fused_kernel,
        grid=(num_blocks,),
        in_specs=[
            idx_spec, idx_spec, idx_spec,
            pl.BlockSpec((RB, L, _NUM_BET_BINS), lambda i: (i, 0, 0)),
            full_spec((_NUM_BET_BINS, _D)),
            full_spec((1, _D)), full_spec((1, _D)), full_spec((1, _D)),
            full_spec((_PACKED_ROWS, _D + 1)),
        ],
        out_specs=pl.BlockSpec((RB, L, _D), lambda i: (i, 0, 0)),
        out_shape=jax.ShapeDtypeStruct((B, L, _D), jnp.float32),
        compiler_params=pltpu.CompilerParams(
            dimension_semantics=("parallel",)),
    )(tok, act, st, action_legal_masks, mlp_w, mlp_b.reshape(1, _D),
      ln_gamma.reshape(1, _D), ln_beta.reshape(1, _D), packed)

    return out


# speedup vs baseline: 1.0757x; 1.0757x over previous
import jax
import jax.numpy as jnp
from jax.experimental import pallas as pl
from jax.experimental.pallas import tpu as pltpu

_NUM_BET_BINS = 32
_D = 128
_NUM_STREETS = 4
_OFFSET = 10
_PACKED_ROWS = 40
_SEG = 56


def _fused_kernel(tok_ref, act_ref, st_ref, x_ref, w_ref, b_ref, g_ref,
                  be_ref, t_ref, out_ref):
    rb, ll = tok_ref.shape
    p = rb * _SEG

    zpad = jnp.zeros((_SEG - ll, _NUM_BET_BINS), jnp.float32)
    xs = []
    for i in range(rb):
        xs.append(x_ref[i])
        xs.append(zpad)
    x = jnp.concatenate(xs, axis=0)
    h_all = jnp.dot(x, w_ref[...], preferred_element_type=jnp.float32)
    h_all = h_all + b_ref[...]
    h = h_all[:, :_D]
    mu = h_all[:, _D:]
    d = h - mu
    var = jnp.mean(d * d, axis=-1, keepdims=True)
    hn = d * jax.lax.rsqrt(var + 1e-5) * g_ref[...] + be_ref[...]
    hr = jnp.maximum(hn, 0.0)

    ipad = jnp.full((1, _SEG - ll), -1, jnp.int32)
    cat = lambda r: jnp.concatenate(
        [q for i in range(rb) for q in (r[i:i + 1, :], ipad)], axis=1)
    tok = cat(tok_ref)
    mask = (tok >= _OFFSET) & (tok < _OFFSET + _NUM_BET_BINS)
    a = jnp.where(mask, jnp.clip(cat(act_ref), 0, 1), -1)
    s = jnp.where(mask, jnp.clip(cat(st_ref), 0, _NUM_STREETS - 1) + 2, -1)
    t = jnp.where(mask, jnp.clip(tok - _OFFSET, 0, _NUM_BET_BINS - 1) + 6, -1)

    sub = jax.lax.broadcasted_iota(jnp.int32, (_PACKED_ROWS, p), 0)
    oh = jnp.where((sub == a) | (sub == s) | (sub == t), 1.0, 0.0)
    ea = jax.lax.dot_general(oh, t_ref[...], (((0,), (0,)), ((), ())),
                             preferred_element_type=jnp.float32)
    out = ea[:, :_D] + ea[:, _D:] * hr
    for i in range(rb):
        out_ref[i] = out[i * _SEG:i * _SEG + ll, :]


def kernel(token_ids, action_actors, action_streets, action_legal_masks,
           actor_emb_w, street_emb_w, action_type_emb_w, mlp_w, mlp_b,
           ln_gamma, ln_beta):
    B, L = token_ids.shape
    RB = 128
    num_blocks = pl.cdiv(B, RB)

    tok = token_ids.astype(jnp.int32)
    act = action_actors.astype(jnp.int32)
    st = action_streets.astype(jnp.int32)

    wmu = jnp.tile(jnp.mean(mlp_w, axis=1, keepdims=True), (1, _D))
    w_aug = jnp.concatenate([mlp_w, wmu], axis=1)
    b_aug = jnp.concatenate(
        [mlp_b.reshape(1, _D),
         jnp.full((1, _D), jnp.mean(mlp_b), jnp.float32)], axis=1)

    packed = jnp.concatenate([
        actor_emb_w, street_emb_w, action_type_emb_w,
        jnp.zeros((_PACKED_ROWS - 38, _D), jnp.float32)], axis=0)
    ind = jnp.zeros((_PACKED_ROWS, _D), jnp.float32).at[0:2, :].set(1.0)
    packed = jnp.concatenate([packed, ind], axis=1)

    idx_spec = pl.BlockSpec((RB, L), lambda i: (i, 0))
    full_spec = lambda shape: pl.BlockSpec(shape, lambda i: (0,) * len(shape))

    out = pl.pallas_call(
        _fused_kernel,
        grid=(num_blocks,),
        in_specs=[
            idx_spec, idx_spec, idx_spec,
            pl.BlockSpec((RB, L, _NUM_BET_BINS), lambda i: (i, 0, 0)),
            full_spec((_NUM_BET_BINS, 2 * _D)),
            full_spec((1, 2 * _D)), full_spec((1, _D)), full_spec((1, _D)),
            full_spec((_PACKED_ROWS, 2 * _D)),
        ],
        out_specs=pl.BlockSpec((RB, L, _D), lambda i: (i, 0, 0)),
        out_shape=jax.ShapeDtypeStruct((B, L, _D), jnp.float32),
        compiler_params=pltpu.CompilerParams(
            dimension_semantics=("parallel",)),
    )(tok, act, st, action_legal_masks, w_aug, b_aug,
      ln_gamma.reshape(1, _D), ln_beta.reshape(1, _D), packed)

    return out


# speedup vs baseline: 1.0777x; 1.0019x over previous
import jax
import jax.numpy as jnp
from jax.experimental import pallas as pl
from jax.experimental.pallas import tpu as pltpu

_NUM_BET_BINS = 32
_D = 128
_NUM_STREETS = 4
_OFFSET = 10
_PACKED_ROWS = 40
_SEG = 56


def _fused_kernel(tok_ref, act_ref, st_ref, x_hbm, w_ref, b_ref, g_ref,
                  be_ref, t_ref, out_ref, xbuf, sem):
    rb, ll = tok_ref.shape
    p = rb * _SEG

    gi = pl.program_id(0)
    nb = pl.num_programs(0)
    slot = jax.lax.rem(gi, 2)
    nxt = jax.lax.rem(gi + 1, 2)

    def _copy(blk, sl):
        return pltpu.make_async_copy(
            x_hbm.at[pl.ds(blk * rb, rb)], xbuf.at[sl], sem.at[sl])

    @pl.when(gi == 0)
    def _():
        _copy(gi, slot).start()

    @pl.when(gi + 1 < nb)
    def _():
        _copy(gi + 1, nxt).start()

    _copy(gi, slot).wait()
    x_ref = xbuf.at[slot]

    zpad = jnp.zeros((_SEG - ll, _NUM_BET_BINS), jnp.float32)
    xs = []
    for i in range(rb):
        xs.append(x_ref[i])
        xs.append(zpad)
    x = jnp.concatenate(xs, axis=0)
    h_all = jnp.dot(x, w_ref[...], preferred_element_type=jnp.float32)
    h_all = h_all + b_ref[...]
    h = h_all[:, :_D]
    mu = h_all[:, _D:]
    d = h - mu
    var = jnp.mean(d * d, axis=-1, keepdims=True)
    hn = d * jax.lax.rsqrt(var + 1e-5) * g_ref[...] + be_ref[...]
    hr = jnp.maximum(hn, 0.0)

    ipad = jnp.full((1, _SEG - ll), -1, jnp.int32)
    cat = lambda r: jnp.concatenate(
        [q for i in range(rb) for q in (r[i:i + 1, :], ipad)], axis=1)
    tok = cat(tok_ref)
    mask = (tok >= _OFFSET) & (tok < _OFFSET + _NUM_BET_BINS)
    a = jnp.where(mask, jnp.clip(cat(act_ref), 0, 1), -1)
    s = jnp.where(mask, jnp.clip(cat(st_ref), 0, _NUM_STREETS - 1) + 2, -1)
    t = jnp.where(mask, jnp.clip(tok - _OFFSET, 0, _NUM_BET_BINS - 1) + 6, -1)

    sub = jax.lax.broadcasted_iota(jnp.int32, (_PACKED_ROWS, p), 0)
    oh = jnp.where((sub == a) | (sub == s) | (sub == t), 1.0, 0.0)
    ea = jax.lax.dot_general(oh, t_ref[...], (((0,), (0,)), ((), ())),
                             preferred_element_type=jnp.float32)
    out = ea[:, :_D] + ea[:, _D:] * hr
    for i in range(rb):
        out_ref[i] = out[i * _SEG:i * _SEG + ll, :]


def kernel(token_ids, action_actors, action_streets, action_legal_masks,
           actor_emb_w, street_emb_w, action_type_emb_w, mlp_w, mlp_b,
           ln_gamma, ln_beta):
    B, L = token_ids.shape
    RB = 128
    num_blocks = pl.cdiv(B, RB)

    tok = token_ids.astype(jnp.int32)
    act = action_actors.astype(jnp.int32)
    st = action_streets.astype(jnp.int32)

    wmu = jnp.tile(jnp.mean(mlp_w, axis=1, keepdims=True), (1, _D))
    w_aug = jnp.concatenate([mlp_w, wmu], axis=1)
    b_aug = jnp.concatenate(
        [mlp_b.reshape(1, _D),
         jnp.full((1, _D), jnp.mean(mlp_b), jnp.float32)], axis=1)

    packed = jnp.concatenate([
        actor_emb_w, street_emb_w, action_type_emb_w,
        jnp.zeros((_PACKED_ROWS - 38, _D), jnp.float32)], axis=0)
    ind = jnp.zeros((_PACKED_ROWS, _D), jnp.float32).at[0:2, :].set(1.0)
    packed = jnp.concatenate([packed, ind], axis=1)

    idx_spec = pl.BlockSpec((RB, L), lambda i: (i, 0))
    full_spec = lambda shape: pl.BlockSpec(shape, lambda i: (0,) * len(shape))

    out = pl.pallas_call(
        _fused_kernel,
        grid=(num_blocks,),
        in_specs=[
            idx_spec, idx_spec, idx_spec,
            pl.BlockSpec(memory_space=pl.ANY),
            full_spec((_NUM_BET_BINS, 2 * _D)),
            full_spec((1, 2 * _D)), full_spec((1, _D)), full_spec((1, _D)),
            full_spec((_PACKED_ROWS, 2 * _D)),
        ],
        out_specs=pl.BlockSpec((RB, L, _D), lambda i: (i, 0, 0)),
        out_shape=jax.ShapeDtypeStruct((B, L, _D), jnp.float32),
        scratch_shapes=[
            pltpu.VMEM((2, RB, L, _NUM_BET_BINS), jnp.float32),
            pltpu.SemaphoreType.DMA((2,)),
        ],
        compiler_params=pltpu.CompilerParams(
            dimension_semantics=("parallel",)),
    )(tok, act, st, action_legal_masks, w_aug, b_aug,
      ln_gamma.reshape(1, _D), ln_beta.reshape(1, _D), packed)

    return out
